# Initial kernel scaffold; baseline (speedup 1.0000x reference)
import jax
import jax.numpy as jnp
from jax.experimental import pallas as pl


def kernel(x, edge_index, edge_attr, W1, b1, W2, b2, W3, b3, W4, b4):
    raise NotImplementedError("write your pallas kernel here")


# speedup vs baseline: 4.1543x; 4.1543x over previous
import functools

import jax
import jax.numpy as jnp
from jax import lax
from jax.experimental import pallas as pl
from jax.experimental.pallas import tpu as pltpu
from jax.experimental.pallas import tpu_sc as plsc

N, E, D, DE, H = 10000, 320000, 128, 32, 128

NC, NS = 2, 16
NW = NC * NS
EB = 128
BLOCKS_PER_TILE = -(-E // (NW * EB))
E_PER_TILE = BLOCKS_PER_TILE * EB
E_PAD = E_PER_TILE * NW
N_PAD = 10240
ROWS_PER_TILE = N_PAD // NS

_mesh = plsc.VectorSubcoreMesh(core_axis_name="c", subcore_axis_name="s")


def _pq_body(x_ref, wa_ref, wb_ref, p_ref, q_ref):
    xb = x_ref[...]
    p_ref[...] = jnp.dot(xb, wa_ref[...], preferred_element_type=jnp.float32)
    q_ref[...] = jnp.dot(xb, wb_ref[...], preferred_element_type=jnp.float32)


def _pq(x, wa, wb):
    bn = 2000
    grid = (N // bn,)
    return pl.pallas_call(
        _pq_body,
        grid=grid,
        in_specs=[
            pl.BlockSpec((bn, D), lambda i: (i, 0)),
            pl.BlockSpec((D, H), lambda i: (0, 0)),
            pl.BlockSpec((D, H), lambda i: (0, 0)),
        ],
        out_specs=[
            pl.BlockSpec((bn, H), lambda i: (i, 0)),
            pl.BlockSpec((bn, H), lambda i: (i, 0)),
        ],
        out_shape=[
            jax.ShapeDtypeStruct((N, H), jnp.float32),
            jax.ShapeDtypeStruct((N, H), jnp.float32),
        ],
    )(x, wa, wb)


@functools.partial(
    pl.kernel,
    mesh=_mesh,
    out_type=jax.ShapeDtypeStruct((E_PAD, H), jnp.float32),
    scratch_types=[
        pltpu.VMEM((EB,), jnp.int32),
        pltpu.VMEM((EB,), jnp.int32),
        pltpu.VMEM((EB, H), jnp.float32),
        pltpu.VMEM((EB, H), jnp.float32),
        pltpu.SemaphoreType.DMA,
        pltpu.SemaphoreType.DMA,
    ],
)
def _gather_add(p_hbm, q_hbm, src_hbm, dst_hbm, g_hbm,
                idx_s, idx_d, bufa, bufb, sema, semb):
    c = lax.axis_index("c")
    s = lax.axis_index("s")
    base = (c * NS + s) * E_PER_TILE

    def block(i, carry):
        off = base + i * EB
        pltpu.sync_copy(src_hbm.at[pl.ds(off, EB)], idx_s)
        pltpu.sync_copy(dst_hbm.at[pl.ds(off, EB)], idx_d)
        cpa = pltpu.async_copy(p_hbm.at[idx_s], bufa, sema)
        cpb = pltpu.async_copy(q_hbm.at[idx_d], bufb, semb)
        cpa.wait()
        cpb.wait()

        def row(r, rc):
            for k in range(H // 16):
                sl = pl.ds(k * 16, 16)
                plsc.addupdate(bufa.at[r, sl], bufb[r, sl])
            return rc

        lax.fori_loop(0, EB, row, 0, unroll=2)
        pltpu.sync_copy(bufa, g_hbm.at[pl.ds(off, EB)])
        return carry

    lax.fori_loop(0, BLOCKS_PER_TILE, block, 0)


def _mlp_body(g_ref, ea_ref, w1c_ref, b1_ref, w2_ref, b2_ref,
              w3_ref, b3_ref, w4_ref, b4_ref, o_ref):
    h = (g_ref[...]
         + jnp.dot(ea_ref[...], w1c_ref[...], preferred_element_type=jnp.float32)
         + b1_ref[...])
    h = jnp.maximum(h, 0.0)
    h = jnp.maximum(
        jnp.dot(h, w2_ref[...], preferred_element_type=jnp.float32) + b2_ref[...], 0.0)
    h = jnp.maximum(
        jnp.dot(h, w3_ref[...], preferred_element_type=jnp.float32) + b3_ref[...], 0.0)
    o_ref[...] = jnp.dot(h, w4_ref[...], preferred_element_type=jnp.float32) + b4_ref[...]


def _mlp(g, ea, w1c, b1, w2, b2, w3, b3, w4, b4):
    be = 2048
    grid = (E_PAD // be,)
    full = lambda shape: pl.BlockSpec(shape, lambda i: tuple(0 for _ in shape))
    return pl.pallas_call(
        _mlp_body,
        grid=grid,
        in_specs=[
            pl.BlockSpec((be, H), lambda i: (i, 0)),
            pl.BlockSpec((be, DE), lambda i: (i, 0)),
            full((DE, H)),
            full((1, H)),
            full((H, H)),
            full((1, H)),
            full((H, H)),
            full((1, H)),
            full((H, H)),
            full((1, H)),
        ],
        out_specs=pl.BlockSpec((be, H), lambda i: (i, 0)),
        out_shape=jax.ShapeDtypeStruct((E_PAD, H), jnp.float32),
    )(g, ea, w1c, b1, w2, b2, w3, b3, w4, b4)


@functools.partial(
    pl.kernel,
    mesh=_mesh,
    out_type=jax.ShapeDtypeStruct((NC, N_PAD, H), jnp.float32),
    scratch_types=[
        pltpu.VMEM((EB,), jnp.int32),
        pltpu.VMEM((EB, H), jnp.float32),
        pltpu.VMEM((EB, H), jnp.float32),
        pltpu.VMEM_SHARED((N_PAD, H), jnp.float32),
    ],
)
def _scatter_add(msg_hbm, dst_hbm, part_hbm, idx_d, mbuf, zbuf, acc):
    c = lax.axis_index("c")
    s = lax.axis_index("s")
    base = (c * NS + s) * E_PER_TILE

    def zrow(r, carry):
        for k in range(H // 16):
            zbuf[r, pl.ds(k * 16, 16)] = jnp.zeros((16,), jnp.float32)
        return carry

    lax.fori_loop(0, EB, zrow, 0)

    def zchunk(j, carry):
        pltpu.sync_copy(zbuf, acc.at[pl.ds(s * ROWS_PER_TILE + j * EB, EB)])
        return carry

    lax.fori_loop(0, ROWS_PER_TILE // EB, zchunk, 0)
    rows = pl.ds(s * ROWS_PER_TILE, ROWS_PER_TILE)
    plsc.subcore_barrier()

    def block(i, carry):
        off = base + i * EB
        pltpu.sync_copy(dst_hbm.at[pl.ds(off, EB)], idx_d)
        pltpu.sync_copy(msg_hbm.at[pl.ds(off, EB)], mbuf)
        pltpu.sync_copy(mbuf, acc.at[idx_d], add=True)
        return carry

    lax.fori_loop(0, BLOCKS_PER_TILE, block, 0)
    plsc.subcore_barrier()
    pltpu.sync_copy(acc.at[rows], part_hbm.at[c, rows])


def _combine_body(a_ref, b_ref, o_ref):
    o_ref[...] = a_ref[...] + b_ref[...]


def _combine(p0, p1):
    bn = 2000
    grid = (N // bn,)
    return pl.pallas_call(
        _combine_body,
        grid=grid,
        in_specs=[
            pl.BlockSpec((bn, H), lambda i: (i, 0)),
            pl.BlockSpec((bn, H), lambda i: (i, 0)),
        ],
        out_specs=pl.BlockSpec((bn, H), lambda i: (i, 0)),
        out_shape=jax.ShapeDtypeStruct((N, H), jnp.float32),
    )(p0, p1)


def kernel(x, edge_index, edge_attr, W1, b1, W2, b2, W3, b3, W4, b4):
    pad = E_PAD - E
    src = edge_index[:, 0].astype(jnp.int32)
    dst = edge_index[:, 1].astype(jnp.int32)
    src_p = jnp.concatenate([src, jnp.zeros((pad,), jnp.int32)])
    dst_p = jnp.concatenate([dst, jnp.full((pad,), N, jnp.int32)])
    ea_p = jnp.concatenate([edge_attr, jnp.zeros((pad, DE), edge_attr.dtype)])

    w1a, w1b, w1c = W1[:D], W1[D:2 * D], W1[2 * D:]
    p, q = _pq(x, w1a, w1b)
    g = _gather_add(p, q, src_p, dst_p)
    msg = _mlp(g, ea_p, w1c, b1.reshape(1, H), W2, b2.reshape(1, H),
               W3, b3.reshape(1, H), W4, b4.reshape(1, H))
    part = _scatter_add(msg, dst_p)
    return _combine(part[0, :N], part[1, :N])


# speedup vs baseline: 4.4280x; 1.0659x over previous
import functools

import jax
import jax.numpy as jnp
from jax import lax
from jax.experimental import pallas as pl
from jax.experimental.pallas import tpu as pltpu
from jax.experimental.pallas import tpu_sc as plsc

N, E, D, DE, H = 10000, 320000, 128, 32, 128

NC, NS = 2, 16
NW = NC * NS
EB = 128
BLOCKS_PER_TILE = -(-E // (NW * EB))
E_PER_TILE = BLOCKS_PER_TILE * EB
E_PAD = E_PER_TILE * NW
N_PAD = 10240
ROWS_PER_TILE = N_PAD // NS

_mesh = plsc.VectorSubcoreMesh(core_axis_name="c", subcore_axis_name="s")


def _pq_body(x_ref, wa_ref, wb_ref, p_ref, q_ref):
    xb = x_ref[...]
    p_ref[...] = jnp.dot(xb, wa_ref[...], preferred_element_type=jnp.float32)
    q_ref[...] = jnp.dot(xb, wb_ref[...], preferred_element_type=jnp.float32)


def _pq(x, wa, wb):
    bn = 2000
    grid = (N // bn,)
    return pl.pallas_call(
        _pq_body,
        grid=grid,
        in_specs=[
            pl.BlockSpec((bn, D), lambda i: (i, 0)),
            pl.BlockSpec((D, H), lambda i: (0, 0)),
            pl.BlockSpec((D, H), lambda i: (0, 0)),
        ],
        out_specs=[
            pl.BlockSpec((bn, H), lambda i: (i, 0)),
            pl.BlockSpec((bn, H), lambda i: (i, 0)),
        ],
        out_shape=[
            jax.ShapeDtypeStruct((N, H), jnp.float32),
            jax.ShapeDtypeStruct((N, H), jnp.float32),
        ],
    )(x, wa, wb)


@functools.partial(
    pl.kernel,
    mesh=_mesh,
    out_type=jax.ShapeDtypeStruct((E_PAD, H), jnp.float32),
    scratch_types=[
        pltpu.VMEM((E_PER_TILE,), jnp.int32),
        pltpu.VMEM((E_PER_TILE,), jnp.int32),
        pltpu.VMEM((EB,), jnp.int32),
        pltpu.VMEM((EB,), jnp.int32),
        pltpu.VMEM((EB, H), jnp.float32),
        pltpu.VMEM((EB, H), jnp.float32),
        pltpu.SemaphoreType.DMA,
        pltpu.SemaphoreType.DMA,
    ],
)
def _gather_add(p_hbm, q_hbm, src_hbm, dst_hbm, g_hbm,
                idx_sall, idx_dall, idx_s, idx_d, bufa, bufb, sema, semb):
    c = lax.axis_index("c")
    s = lax.axis_index("s")
    base = (c * NS + s) * E_PER_TILE

    boff = pl.multiple_of(base, 8)
    pltpu.sync_copy(src_hbm.at[pl.ds(boff, E_PER_TILE)], idx_sall)
    pltpu.sync_copy(dst_hbm.at[pl.ds(boff, E_PER_TILE)], idx_dall)

    def block(i, carry):
        off = base + i * EB
        for k in range(EB // 16):
            sl = pl.ds(k * 16, 16)
            idx_s[sl] = idx_sall[pl.ds(i * EB + k * 16, 16)]
            idx_d[sl] = idx_dall[pl.ds(i * EB + k * 16, 16)]
        cpa = pltpu.async_copy(p_hbm.at[idx_s], bufa, sema)
        cpb = pltpu.async_copy(q_hbm.at[idx_d], bufb, semb)
        cpa.wait()
        cpb.wait()

        def row(r, rc):
            for k in range(H // 16):
                sl = pl.ds(k * 16, 16)
                plsc.addupdate(bufa.at[r, sl], bufb[r, sl])
            return rc

        lax.fori_loop(0, EB, row, 0, unroll=2)
        pltpu.sync_copy(bufa, g_hbm.at[pl.ds(off, EB)])
        return carry

    lax.fori_loop(0, BLOCKS_PER_TILE, block, 0)


def _mlp_body(g_ref, ea_ref, w1c_ref, b1_ref, w2_ref, b2_ref,
              w3_ref, b3_ref, w4_ref, b4_ref, o_ref):
    h = (g_ref[...]
         + jnp.dot(ea_ref[...], w1c_ref[...], preferred_element_type=jnp.float32)
         + b1_ref[...])
    h = jnp.maximum(h, 0.0)
    h = jnp.maximum(
        jnp.dot(h, w2_ref[...], preferred_element_type=jnp.float32) + b2_ref[...], 0.0)
    h = jnp.maximum(
        jnp.dot(h, w3_ref[...], preferred_element_type=jnp.float32) + b3_ref[...], 0.0)
    o_ref[...] = jnp.dot(h, w4_ref[...], preferred_element_type=jnp.float32) + b4_ref[...]


def _mlp(g, ea, w1c, b1, w2, b2, w3, b3, w4, b4):
    be = 2048
    grid = (E_PAD // be,)
    full = lambda shape: pl.BlockSpec(shape, lambda i: tuple(0 for _ in shape))
    return pl.pallas_call(
        _mlp_body,
        grid=grid,
        in_specs=[
            pl.BlockSpec((be, H), lambda i: (i, 0)),
            pl.BlockSpec((be, DE), lambda i: (i, 0)),
            full((DE, H)),
            full((1, H)),
            full((H, H)),
            full((1, H)),
            full((H, H)),
            full((1, H)),
            full((H, H)),
            full((1, H)),
        ],
        out_specs=pl.BlockSpec((be, H), lambda i: (i, 0)),
        out_shape=jax.ShapeDtypeStruct((E_PAD, H), jnp.float32),
    )(g, ea, w1c, b1, w2, b2, w3, b3, w4, b4)


@functools.partial(
    pl.kernel,
    mesh=_mesh,
    out_type=jax.ShapeDtypeStruct((NC, N_PAD, H), jnp.float32),
    scratch_types=[
        pltpu.VMEM((EB,), jnp.int32),
        pltpu.VMEM((EB, H), jnp.float32),
        pltpu.VMEM((EB, H), jnp.float32),
        pltpu.VMEM_SHARED((N_PAD, H), jnp.float32),
    ],
)
def _scatter_add(msg_hbm, dst_hbm, part_hbm, idx_d, mbuf, zbuf, acc):
    c = lax.axis_index("c")
    s = lax.axis_index("s")
    base = (c * NS + s) * E_PER_TILE

    def zrow(r, carry):
        for k in range(H // 16):
            zbuf[r, pl.ds(k * 16, 16)] = jnp.zeros((16,), jnp.float32)
        return carry

    lax.fori_loop(0, EB, zrow, 0)

    def zchunk(j, carry):
        pltpu.sync_copy(zbuf, acc.at[pl.ds(s * ROWS_PER_TILE + j * EB, EB)])
        return carry

    lax.fori_loop(0, ROWS_PER_TILE // EB, zchunk, 0)
    rows = pl.ds(s * ROWS_PER_TILE, ROWS_PER_TILE)
    plsc.subcore_barrier()

    def block(i, carry):
        off = base + i * EB
        pltpu.sync_copy(dst_hbm.at[pl.ds(off, EB)], idx_d)
        pltpu.sync_copy(msg_hbm.at[pl.ds(off, EB)], mbuf)
        pltpu.sync_copy(mbuf, acc.at[idx_d], add=True)
        return carry

    lax.fori_loop(0, BLOCKS_PER_TILE, block, 0)
    plsc.subcore_barrier()
    pltpu.sync_copy(acc.at[rows], part_hbm.at[c, rows])


def _combine_body(a_ref, b_ref, o_ref):
    o_ref[...] = a_ref[...] + b_ref[...]


def _combine(p0, p1):
    bn = 2000
    grid = (N // bn,)
    return pl.pallas_call(
        _combine_body,
        grid=grid,
        in_specs=[
            pl.BlockSpec((bn, H), lambda i: (i, 0)),
            pl.BlockSpec((bn, H), lambda i: (i, 0)),
        ],
        out_specs=pl.BlockSpec((bn, H), lambda i: (i, 0)),
        out_shape=jax.ShapeDtypeStruct((N, H), jnp.float32),
    )(p0, p1)


def kernel(x, edge_index, edge_attr, W1, b1, W2, b2, W3, b3, W4, b4):
    pad = E_PAD - E
    src = edge_index[:, 0].astype(jnp.int32)
    dst = edge_index[:, 1].astype(jnp.int32)
    src_p = jnp.concatenate([src, jnp.zeros((pad,), jnp.int32)])
    dst_p = jnp.concatenate([dst, jnp.full((pad,), N, jnp.int32)])
    ea_p = jnp.concatenate([edge_attr, jnp.zeros((pad, DE), edge_attr.dtype)])

    w1a, w1b, w1c = W1[:D], W1[D:2 * D], W1[2 * D:]
    p, q = _pq(x, w1a, w1b)
    g = _gather_add(p, q, src_p, dst_p)
    msg = _mlp(g, ea_p, w1c, b1.reshape(1, H), W2, b2.reshape(1, H),
               W3, b3.reshape(1, H), W4, b4.reshape(1, H))
    part = _scatter_add(msg, dst_p)
    return _combine(part[0, :N], part[1, :N])


# speedup vs baseline: 4.7431x; 1.0712x over previous
import functools

import jax
import jax.numpy as jnp
from jax import lax
from jax.experimental import pallas as pl
from jax.experimental.pallas import tpu as pltpu
from jax.experimental.pallas import tpu_sc as plsc

N, E, D, DE, H = 10000, 320000, 128, 32, 128

NC, NS = 2, 16
NW = NC * NS
EB = 128
BLOCKS_PER_TILE = -(-E // (NW * EB))
E_PER_TILE = BLOCKS_PER_TILE * EB
E_PAD = E_PER_TILE * NW
N_PAD = 10240
ROWS_PER_TILE = N_PAD // NS

_mesh = plsc.VectorSubcoreMesh(core_axis_name="c", subcore_axis_name="s")


def _pq_body(x_ref, wa_ref, wb_ref, p_ref, q_ref):
    xb = x_ref[...]
    p_ref[...] = jnp.dot(xb, wa_ref[...], preferred_element_type=jnp.float32)
    q_ref[...] = jnp.dot(xb, wb_ref[...], preferred_element_type=jnp.float32)


def _pq(x, wa, wb):
    bn = 2000
    grid = (N // bn,)
    return pl.pallas_call(
        _pq_body,
        grid=grid,
        in_specs=[
            pl.BlockSpec((bn, D), lambda i: (i, 0)),
            pl.BlockSpec((D, H), lambda i: (0, 0)),
            pl.BlockSpec((D, H), lambda i: (0, 0)),
        ],
        out_specs=[
            pl.BlockSpec((bn, H), lambda i: (i, 0)),
            pl.BlockSpec((bn, H), lambda i: (i, 0)),
        ],
        out_shape=[
            jax.ShapeDtypeStruct((N, H), jnp.float32),
            jax.ShapeDtypeStruct((N, H), jnp.float32),
        ],
    )(x, wa, wb)


@functools.partial(
    pl.kernel,
    mesh=_mesh,
    out_type=jax.ShapeDtypeStruct((E_PAD, H), jnp.float32),
    scratch_types=[
        pltpu.VMEM((E_PER_TILE,), jnp.int32),
        pltpu.VMEM((E_PER_TILE,), jnp.int32),
        pltpu.VMEM((EB,), jnp.int32),
        pltpu.VMEM((EB,), jnp.int32),
        pltpu.VMEM((2 * EB, H), jnp.float32),
        pltpu.VMEM((2 * EB, H), jnp.float32),
        pltpu.SemaphoreType.DMA,
        pltpu.SemaphoreType.DMA,
        pltpu.SemaphoreType.DMA,
    ],
)
def _gather_add(p_hbm, q_hbm, src_hbm, dst_hbm, g_hbm,
                idx_sall, idx_dall, idx_s, idx_d, bufa, bufb,
                sema, semb, semw):
    c = lax.axis_index("c")
    s = lax.axis_index("s")
    base = (c * NS + s) * E_PER_TILE

    boff = pl.multiple_of(base, 8)
    pltpu.sync_copy(src_hbm.at[pl.ds(boff, E_PER_TILE)], idx_sall)
    pltpu.sync_copy(dst_hbm.at[pl.ds(boff, E_PER_TILE)], idx_dall)

    def half(par):
        return pl.ds(pl.multiple_of(par * EB, EB), EB)

    def stage_idx(i):
        for k in range(EB // 16):
            sl = pl.ds(k * 16, 16)
            idx_s[sl] = idx_sall[pl.ds(i * EB + k * 16, 16)]
            idx_d[sl] = idx_dall[pl.ds(i * EB + k * 16, 16)]

    def issue_gather(par):
        pltpu.async_copy(p_hbm.at[idx_s], bufa.at[half(par)], sema)
        pltpu.async_copy(q_hbm.at[idx_d], bufb.at[half(par)], semb)

    def wait_gather(par):
        pltpu.make_async_copy(p_hbm.at[idx_s], bufa.at[half(par)], sema).wait()
        pltpu.make_async_copy(q_hbm.at[idx_d], bufb.at[half(par)], semb).wait()

    def drain_write(par, i):
        pltpu.make_async_copy(
            bufa.at[half(par)], g_hbm.at[pl.ds(base + i * EB, EB)],
            semw).wait()

    stage_idx(0)
    issue_gather(0)

    def block(i, carry):
        par = lax.rem(i, 2)
        wait_gather(par)

        @pl.when(i >= 1)
        def _():
            drain_write(1 - par, i - 1)

        @pl.when(i + 1 < BLOCKS_PER_TILE)
        def _():
            stage_idx(i + 1)
            issue_gather(1 - par)

        bo = pl.multiple_of(par * EB, EB)

        def row(r, rc):
            for k in range(H // 16):
                sl = pl.ds(k * 16, 16)
                plsc.addupdate(bufa.at[bo + r, sl], bufb[bo + r, sl])
            return rc

        lax.fori_loop(0, EB, row, 0, unroll=2)
        pltpu.async_copy(
            bufa.at[half(par)], g_hbm.at[pl.ds(base + i * EB, EB)], semw)
        return carry

    lax.fori_loop(0, BLOCKS_PER_TILE, block, 0)
    drain_write((BLOCKS_PER_TILE - 1) % 2, BLOCKS_PER_TILE - 1)


def _mlp_body(g_ref, ea_ref, w1c_ref, b1_ref, w2_ref, b2_ref,
              w3_ref, b3_ref, w4_ref, b4_ref, o_ref):
    h = (g_ref[...]
         + jnp.dot(ea_ref[...], w1c_ref[...], preferred_element_type=jnp.float32)
         + b1_ref[...])
    h = jnp.maximum(h, 0.0)
    h = jnp.maximum(
        jnp.dot(h, w2_ref[...], preferred_element_type=jnp.float32) + b2_ref[...], 0.0)
    h = jnp.maximum(
        jnp.dot(h, w3_ref[...], preferred_element_type=jnp.float32) + b3_ref[...], 0.0)
    o_ref[...] = jnp.dot(h, w4_ref[...], preferred_element_type=jnp.float32) + b4_ref[...]


def _mlp(g, ea, w1c, b1, w2, b2, w3, b3, w4, b4):
    be = 2048
    grid = (E_PAD // be,)
    full = lambda shape: pl.BlockSpec(shape, lambda i: tuple(0 for _ in shape))
    return pl.pallas_call(
        _mlp_body,
        grid=grid,
        in_specs=[
            pl.BlockSpec((be, H), lambda i: (i, 0)),
            pl.BlockSpec((be, DE), lambda i: (i, 0)),
            full((DE, H)),
            full((1, H)),
            full((H, H)),
            full((1, H)),
            full((H, H)),
            full((1, H)),
            full((H, H)),
            full((1, H)),
        ],
        out_specs=pl.BlockSpec((be, H), lambda i: (i, 0)),
        out_shape=jax.ShapeDtypeStruct((E_PAD, H), jnp.float32),
    )(g, ea, w1c, b1, w2, b2, w3, b3, w4, b4)


@functools.partial(
    pl.kernel,
    mesh=_mesh,
    out_type=jax.ShapeDtypeStruct((NC, N_PAD, H), jnp.float32),
    scratch_types=[
        pltpu.VMEM((EB,), jnp.int32),
        pltpu.VMEM((EB, H), jnp.float32),
        pltpu.VMEM((EB, H), jnp.float32),
        pltpu.VMEM_SHARED((N_PAD, H), jnp.float32),
    ],
)
def _scatter_add(msg_hbm, dst_hbm, part_hbm, idx_d, mbuf, zbuf, acc):
    c = lax.axis_index("c")
    s = lax.axis_index("s")
    base = (c * NS + s) * E_PER_TILE

    def zrow(r, carry):
        for k in range(H // 16):
            zbuf[r, pl.ds(k * 16, 16)] = jnp.zeros((16,), jnp.float32)
        return carry

    lax.fori_loop(0, EB, zrow, 0)

    def zchunk(j, carry):
        pltpu.sync_copy(zbuf, acc.at[pl.ds(s * ROWS_PER_TILE + j * EB, EB)])
        return carry

    lax.fori_loop(0, ROWS_PER_TILE // EB, zchunk, 0)
    rows = pl.ds(s * ROWS_PER_TILE, ROWS_PER_TILE)
    plsc.subcore_barrier()

    def block(i, carry):
        off = base + i * EB
        pltpu.sync_copy(dst_hbm.at[pl.ds(off, EB)], idx_d)
        pltpu.sync_copy(msg_hbm.at[pl.ds(off, EB)], mbuf)
        pltpu.sync_copy(mbuf, acc.at[idx_d], add=True)
        return carry

    lax.fori_loop(0, BLOCKS_PER_TILE, block, 0)
    plsc.subcore_barrier()
    pltpu.sync_copy(acc.at[rows], part_hbm.at[c, rows])


def _combine_body(a_ref, b_ref, o_ref):
    o_ref[...] = a_ref[...] + b_ref[...]


def _combine(p0, p1):
    bn = 2000
    grid = (N // bn,)
    return pl.pallas_call(
        _combine_body,
        grid=grid,
        in_specs=[
            pl.BlockSpec((bn, H), lambda i: (i, 0)),
            pl.BlockSpec((bn, H), lambda i: (i, 0)),
        ],
        out_specs=pl.BlockSpec((bn, H), lambda i: (i, 0)),
        out_shape=jax.ShapeDtypeStruct((N, H), jnp.float32),
    )(p0, p1)


def kernel(x, edge_index, edge_attr, W1, b1, W2, b2, W3, b3, W4, b4):
    pad = E_PAD - E
    src = edge_index[:, 0].astype(jnp.int32)
    dst = edge_index[:, 1].astype(jnp.int32)
    src_p = jnp.concatenate([src, jnp.zeros((pad,), jnp.int32)])
    dst_p = jnp.concatenate([dst, jnp.full((pad,), N, jnp.int32)])
    ea_p = jnp.concatenate([edge_attr, jnp.zeros((pad, DE), edge_attr.dtype)])

    w1a, w1b, w1c = W1[:D], W1[D:2 * D], W1[2 * D:]
    p, q = _pq(x, w1a, w1b)
    g = _gather_add(p, q, src_p, dst_p)
    msg = _mlp(g, ea_p, w1c, b1.reshape(1, H), W2, b2.reshape(1, H),
               W3, b3.reshape(1, H), W4, b4.reshape(1, H))
    part = _scatter_add(msg, dst_p)
    return _combine(part[0, :N], part[1, :N])


# speedup vs baseline: 5.1537x; 1.0866x over previous
import functools

import jax
import jax.numpy as jnp
from jax import lax
from jax.experimental import pallas as pl
from jax.experimental.pallas import tpu as pltpu
from jax.experimental.pallas import tpu_sc as plsc

N, E, D, DE, H = 10000, 320000, 128, 32, 128

NC, NS = 2, 16
NW = NC * NS
EB = 128
BLOCKS_PER_TILE = -(-E // (NW * EB))
E_PER_TILE = BLOCKS_PER_TILE * EB
E_PAD = E_PER_TILE * NW
N_PAD = 10240
ROWS_PER_TILE = N_PAD // NS

_mesh = plsc.VectorSubcoreMesh(core_axis_name="c", subcore_axis_name="s")


def _pq_body(x_ref, wa_ref, wb_ref, p_ref, q_ref):
    xb = x_ref[...]
    p_ref[...] = jnp.dot(xb, wa_ref[...], preferred_element_type=jnp.float32)
    q_ref[...] = jnp.dot(xb, wb_ref[...], preferred_element_type=jnp.float32)


def _pq(x, wa, wb):
    bn = 2000
    grid = (N // bn,)
    return pl.pallas_call(
        _pq_body,
        grid=grid,
        in_specs=[
            pl.BlockSpec((bn, D), lambda i: (i, 0)),
            pl.BlockSpec((D, H), lambda i: (0, 0)),
            pl.BlockSpec((D, H), lambda i: (0, 0)),
        ],
        out_specs=[
            pl.BlockSpec((bn, H), lambda i: (i, 0)),
            pl.BlockSpec((bn, H), lambda i: (i, 0)),
        ],
        out_shape=[
            jax.ShapeDtypeStruct((N, H), jnp.float32),
            jax.ShapeDtypeStruct((N, H), jnp.float32),
        ],
    )(x, wa, wb)


@functools.partial(
    pl.kernel,
    mesh=_mesh,
    out_type=jax.ShapeDtypeStruct((E_PAD, H), jnp.float32),
    scratch_types=[
        pltpu.VMEM((E_PER_TILE,), jnp.int32),
        pltpu.VMEM((E_PER_TILE,), jnp.int32),
        pltpu.VMEM((EB,), jnp.int32),
        pltpu.VMEM((EB,), jnp.int32),
        pltpu.VMEM((2 * EB, H), jnp.float32),
        pltpu.VMEM((2 * EB, H), jnp.float32),
        pltpu.SemaphoreType.DMA,
        pltpu.SemaphoreType.DMA,
        pltpu.SemaphoreType.DMA,
    ],
)
def _gather_add(p_hbm, q_hbm, src_hbm, dst_hbm, g_hbm,
                idx_sall, idx_dall, idx_s, idx_d, bufa, bufb,
                sema, semb, semw):
    c = lax.axis_index("c")
    s = lax.axis_index("s")
    base = (c * NS + s) * E_PER_TILE

    boff = pl.multiple_of(base, 8)
    pltpu.sync_copy(src_hbm.at[pl.ds(boff, E_PER_TILE)], idx_sall)
    pltpu.sync_copy(dst_hbm.at[pl.ds(boff, E_PER_TILE)], idx_dall)

    def half(par):
        return pl.ds(pl.multiple_of(par * EB, EB), EB)

    def stage_idx(i):
        for k in range(EB // 16):
            sl = pl.ds(k * 16, 16)
            idx_s[sl] = idx_sall[pl.ds(i * EB + k * 16, 16)]
            idx_d[sl] = idx_dall[pl.ds(i * EB + k * 16, 16)]

    def issue_gather(par):
        pltpu.async_copy(p_hbm.at[idx_s], bufa.at[half(par)], sema)
        pltpu.async_copy(q_hbm.at[idx_d], bufb.at[half(par)], semb)

    def wait_gather(par):
        pltpu.make_async_copy(p_hbm.at[idx_s], bufa.at[half(par)], sema).wait()
        pltpu.make_async_copy(q_hbm.at[idx_d], bufb.at[half(par)], semb).wait()

    def drain_write(par, i):
        pltpu.make_async_copy(
            bufa.at[half(par)], g_hbm.at[pl.ds(base + i * EB, EB)],
            semw).wait()

    stage_idx(0)
    issue_gather(0)

    def block(i, carry):
        par = lax.rem(i, 2)
        wait_gather(par)

        @pl.when(i >= 1)
        def _():
            drain_write(1 - par, i - 1)

        @pl.when(i + 1 < BLOCKS_PER_TILE)
        def _():
            stage_idx(i + 1)
            issue_gather(1 - par)

        bo = pl.multiple_of(par * EB, EB)

        def row(r, rc):
            for k in range(H // 16):
                sl = pl.ds(k * 16, 16)
                plsc.addupdate(bufa.at[bo + r, sl], bufb[bo + r, sl])
            return rc

        lax.fori_loop(0, EB, row, 0, unroll=2)
        pltpu.async_copy(
            bufa.at[half(par)], g_hbm.at[pl.ds(base + i * EB, EB)], semw)
        return carry

    lax.fori_loop(0, BLOCKS_PER_TILE, block, 0)
    drain_write((BLOCKS_PER_TILE - 1) % 2, BLOCKS_PER_TILE - 1)


def _mlp_body(g_ref, ea_ref, w1c_ref, b1_ref, w2_ref, b2_ref,
              w3_ref, b3_ref, w4_ref, b4_ref, o_ref):
    h = (g_ref[...]
         + jnp.dot(ea_ref[...], w1c_ref[...], preferred_element_type=jnp.float32)
         + b1_ref[...])
    h = jnp.maximum(h, 0.0)
    h = jnp.maximum(
        jnp.dot(h, w2_ref[...], preferred_element_type=jnp.float32) + b2_ref[...], 0.0)
    h = jnp.maximum(
        jnp.dot(h, w3_ref[...], preferred_element_type=jnp.float32) + b3_ref[...], 0.0)
    o_ref[...] = jnp.dot(h, w4_ref[...], preferred_element_type=jnp.float32) + b4_ref[...]


def _mlp(g, ea, w1c, b1, w2, b2, w3, b3, w4, b4):
    be = 2048
    grid = (E_PAD // be,)
    full = lambda shape: pl.BlockSpec(shape, lambda i: tuple(0 for _ in shape))
    return pl.pallas_call(
        _mlp_body,
        grid=grid,
        in_specs=[
            pl.BlockSpec((be, H), lambda i: (i, 0)),
            pl.BlockSpec((be, DE), lambda i: (i, 0)),
            full((DE, H)),
            full((1, H)),
            full((H, H)),
            full((1, H)),
            full((H, H)),
            full((1, H)),
            full((H, H)),
            full((1, H)),
        ],
        out_specs=pl.BlockSpec((be, H), lambda i: (i, 0)),
        out_shape=jax.ShapeDtypeStruct((E_PAD, H), jnp.float32),
    )(g, ea, w1c, b1, w2, b2, w3, b3, w4, b4)


@functools.partial(
    pl.kernel,
    mesh=_mesh,
    out_type=jax.ShapeDtypeStruct((NC, N_PAD, H), jnp.float32),
    scratch_types=[
        pltpu.VMEM((EB,), jnp.int32),
        pltpu.VMEM((2 * EB, H), jnp.float32),
        pltpu.SemaphoreType.DMA,
        pltpu.VMEM_SHARED((N_PAD, H), jnp.float32),
    ],
)
def _scatter_add(msg_hbm, dst_hbm, part_hbm, idx_d, mbuf, semm, acc):
    c = lax.axis_index("c")
    s = lax.axis_index("s")
    base = (c * NS + s) * E_PER_TILE

    def zrow(r, carry):
        for k in range(H // 16):
            mbuf[r, pl.ds(k * 16, 16)] = jnp.zeros((16,), jnp.float32)
        return carry

    lax.fori_loop(0, EB, zrow, 0)

    def zchunk(j, carry):
        pltpu.sync_copy(mbuf.at[pl.ds(0, EB)],
                        acc.at[pl.ds(s * ROWS_PER_TILE + j * EB, EB)])
        return carry

    lax.fori_loop(0, ROWS_PER_TILE // EB, zchunk, 0)
    rows = pl.ds(s * ROWS_PER_TILE, ROWS_PER_TILE)
    plsc.subcore_barrier()

    def half(par):
        return pl.ds(pl.multiple_of(par * EB, EB), EB)

    def msg_load(i, par):
        pltpu.async_copy(msg_hbm.at[pl.ds(base + i * EB, EB)],
                         mbuf.at[half(par)], semm)

    def msg_wait(i, par):
        pltpu.make_async_copy(msg_hbm.at[pl.ds(base + i * EB, EB)],
                              mbuf.at[half(par)], semm).wait()

    msg_load(0, 0)

    def block(i, carry):
        par = lax.rem(i, 2)
        off = base + i * EB
        pltpu.sync_copy(dst_hbm.at[pl.ds(off, EB)], idx_d)
        msg_wait(i, par)

        @pl.when(i + 1 < BLOCKS_PER_TILE)
        def _():
            msg_load(i + 1, 1 - par)

        pltpu.sync_copy(mbuf.at[half(par)], acc.at[idx_d], add=True)
        return carry

    lax.fori_loop(0, BLOCKS_PER_TILE, block, 0)
    plsc.subcore_barrier()
    pltpu.sync_copy(acc.at[rows], part_hbm.at[c, rows])


def _combine_body(a_ref, b_ref, o_ref):
    o_ref[...] = a_ref[...] + b_ref[...]


def _combine(p0, p1):
    bn = 2000
    grid = (N // bn,)
    return pl.pallas_call(
        _combine_body,
        grid=grid,
        in_specs=[
            pl.BlockSpec((bn, H), lambda i: (i, 0)),
            pl.BlockSpec((bn, H), lambda i: (i, 0)),
        ],
        out_specs=pl.BlockSpec((bn, H), lambda i: (i, 0)),
        out_shape=jax.ShapeDtypeStruct((N, H), jnp.float32),
    )(p0, p1)


def kernel(x, edge_index, edge_attr, W1, b1, W2, b2, W3, b3, W4, b4):
    pad = E_PAD - E
    src = edge_index[:, 0].astype(jnp.int32)
    dst = edge_index[:, 1].astype(jnp.int32)
    src_p = jnp.concatenate([src, jnp.zeros((pad,), jnp.int32)])
    dst_p = jnp.concatenate([dst, jnp.full((pad,), N, jnp.int32)])
    ea_p = jnp.concatenate([edge_attr, jnp.zeros((pad, DE), edge_attr.dtype)])

    w1a, w1b, w1c = W1[:D], W1[D:2 * D], W1[2 * D:]
    p, q = _pq(x, w1a, w1b)
    g = _gather_add(p, q, src_p, dst_p)
    msg = _mlp(g, ea_p, w1c, b1.reshape(1, H), W2, b2.reshape(1, H),
               W3, b3.reshape(1, H), W4, b4.reshape(1, H))
    part = _scatter_add(msg, dst_p)
    return _combine(part[0, :N], part[1, :N])
